# baseline (speedup 1.0000x reference)
import jax
import jax.numpy as jnp
from jax import lax
from jax.experimental import pallas as pl
from jax.experimental.pallas import tpu as pltpu
from jax.experimental.pallas import tpu_sc as plsc

_NC = 2
_NS = 16
_L = 16
_K = 80
_TR = 512
_G = 64
_BN = 1000


def _half(N):
  return -(-N // 16) * 8


def _arows(N):
  return -(-(_half(N) + _TR) // (_NS * 8)) * (_NS * 8)


def _seg_common(N, E):
  epw = E // _NS
  cap = epw + _K
  half = _half(N)
  arows = _arows(N)
  rps = arows // _NS
  return epw, cap, half, arows, rps


def _stream_loop(table_hbm, agg, src_v, dst_v, buf0, buf1, sem0, sem1,
                 ssem0, ssem1, ncc, nloop):
  def gstart(i, buf, sem):
    pltpu.async_copy(table_hbm.at[src_v.at[pl.ds(i * _K, _K)]], buf, sem)

  def gwait(i, buf, sem):
    pltpu.make_async_copy(table_hbm.at[src_v.at[pl.ds(i * _K, _K)]],
                          buf, sem).wait()

  def sstart(i, buf, sem):
    pltpu.async_copy(buf, agg.at[dst_v.at[pl.ds(i * _K, _K)]], sem,
                     add=True)

  def swait(i, buf, sem):
    pltpu.make_async_copy(buf, agg.at[dst_v.at[pl.ds(i * _K, _K)]],
                          sem).wait()

  def step(i, buf, sem, ssem, nbuf, nsem, nssem):
    @pl.when(i >= 1)
    def _():
      swait(i - 1, nbuf, nssem)

    @pl.when(i + 1 < ncc)
    def _():
      gstart(i + 1, nbuf, nsem)
    gwait(i, buf, sem)
    sstart(i, buf, ssem)

  @pl.when(ncc > 0)
  def _():
    gstart(0, buf0, sem0)

  def loop_body(i, carry):
    @pl.when(jnp.logical_and(i < ncc, i % 2 == 0))
    def _():
      step(i, buf0, sem0, ssem0, buf1, sem1, ssem1)

    @pl.when(jnp.logical_and(i < ncc, i % 2 == 1))
    def _():
      step(i, buf1, sem1, ssem1, buf0, sem0, ssem0)
    return carry

  lax.fori_loop(0, nloop, loop_body, 0)

  @pl.when(jnp.logical_and(ncc > 0, (ncc - 1) % 2 == 0))
  def _():
    swait(ncc - 1, buf0, ssem0)

  @pl.when(jnp.logical_and(ncc > 0, (ncc - 1) % 2 == 1))
  def _():
    swait(ncc - 1, buf1, ssem1)


def _make_seg_compact(N, D, E):
  epw, cap, half, arows, rps = _seg_common(N, E)
  assert epw % _K == 0 and _K % _L == 0

  mesh = plsc.VectorSubcoreMesh(core_axis_name="c", subcore_axis_name="s",
                                num_cores=_NC, num_subcores=_NS)

  def body(src_hbm, dst_hbm, table_hbm, zeros_hbm,
           out_hbm, srcc_hbm, dstc_hbm, cntc_hbm,
           src_v, dst_v, cnt_v, buf0, buf1, agg, sem0, sem1, ssem0,
           ssem1):
    c = lax.axis_index("c")
    s = lax.axis_index("s")
    lane = lax.iota(jnp.int32, _L)
    pltpu.sync_copy(zeros_hbm.at[pl.ds(s * rps, rps), :],
                    agg.at[pl.ds(s * rps, rps), :])
    pltpu.sync_copy(src_hbm.at[pl.ds(c * E + s * epw, epw)],
                    src_v.at[pl.ds(0, epw)])
    pltpu.sync_copy(dst_hbm.at[pl.ds(s * epw, epw)],
                    dst_v.at[pl.ds(0, epw)])

    lo = c * half
    def compact(g, cnt):
      d = dst_v[pl.ds(g * _L, _L)] - lo
      ok = jnp.logical_and(d >= 0, d < half)
      oki = jnp.where(ok, 1, 0)
      srcs = src_v[pl.ds(g * _L, _L)]
      packed = (srcs << 13) | jnp.where(ok, d, 0)
      sv = plsc.sort_key_val(oki, packed, descending=True)[1]
      src_v[pl.ds(cnt, _L)] = sv >> 13
      dst_v[pl.ds(cnt, _L)] = sv & ((1 << 13) - 1)
      return cnt + jnp.sum(oki)
    cnt = lax.fori_loop(0, epw // _L, compact, 0)
    ncc = (cnt + _K - 1) // _K

    def tailfix(t, carry):
      src_v[pl.ds(cnt + t * _L, _L)] = jnp.zeros((_L,), jnp.int32)
      dst_v[pl.ds(cnt + t * _L, _L)] = half + t * _L + lane
      return carry
    lax.fori_loop(0, _K // _L, tailfix, 0)

    pltpu.sync_copy(src_v, srcc_hbm.at[c, s])
    pltpu.sync_copy(dst_v, dstc_hbm.at[c, s])
    cnt_v[...] = jnp.full((_L,), cnt, jnp.int32)
    pltpu.sync_copy(cnt_v, cntc_hbm.at[c, s])
    plsc.subcore_barrier()

    _stream_loop(table_hbm, agg, src_v, dst_v, buf0, buf1, sem0, sem1,
                 ssem0, ssem1, ncc, epw // _K)

    plsc.subcore_barrier()
    pltpu.sync_copy(agg.at[pl.ds(s * rps, rps), :],
                    out_hbm.at[c, pl.ds(s * rps, rps), :])

  return pl.kernel(
      body,
      out_type=(
          jax.ShapeDtypeStruct((_NC, arows, D), jnp.float32),
          jax.ShapeDtypeStruct((_NC, _NS, cap), jnp.int32),
          jax.ShapeDtypeStruct((_NC, _NS, cap), jnp.int32),
          jax.ShapeDtypeStruct((_NC, _NS, _L), jnp.int32),
      ),
      mesh=mesh,
      compiler_params=pltpu.CompilerParams(needs_layout_passes=False),
      scratch_types=[
          pltpu.VMEM((cap,), jnp.int32),
          pltpu.VMEM((cap,), jnp.int32),
          pltpu.VMEM((_L,), jnp.int32),
          pltpu.VMEM((_K, D), jnp.float32),
          pltpu.VMEM((_K, D), jnp.float32),
          pltpu.VMEM_SHARED((arows, D), jnp.float32),
          pltpu.SemaphoreType.DMA,
          pltpu.SemaphoreType.DMA,
          pltpu.SemaphoreType.DMA,
          pltpu.SemaphoreType.DMA,
      ],
  )


def _make_seg_reuse2(N, D, E):
  epw, cap, half, arows, rps = _seg_common(N, E)

  mesh = plsc.VectorSubcoreMesh(core_axis_name="c", subcore_axis_name="s",
                                num_cores=_NC, num_subcores=_NS)

  def body(srcc_hbm, dstc_hbm, cntc_hbm, table_hbm, zeros_hbm, out_hbm,
           src_v, dst_v, cnt_v, buf0, buf1, agg, sem0, sem1, ssem0,
           ssem1):
    c = lax.axis_index("c")
    s = lax.axis_index("s")
    pltpu.sync_copy(zeros_hbm.at[pl.ds(s * rps, rps), :],
                    agg.at[pl.ds(s * rps, rps), :])
    pltpu.sync_copy(srcc_hbm.at[c, s], src_v)
    pltpu.sync_copy(dstc_hbm.at[c, s], dst_v)
    pltpu.sync_copy(cntc_hbm.at[c, s], cnt_v)
    cnt = lax.reduce_max(cnt_v[...], (0,))
    ncc = (cnt + _K - 1) // _K
    plsc.subcore_barrier()

    _stream_loop(table_hbm, agg, src_v, dst_v, buf0, buf1, sem0, sem1,
                 ssem0, ssem1, ncc, epw // _K)

    plsc.subcore_barrier()
    pltpu.sync_copy(agg.at[pl.ds(s * rps, rps), :],
                    out_hbm.at[0, c, pl.ds(s * rps, rps), :])
    plsc.subcore_barrier()

    pltpu.sync_copy(zeros_hbm.at[pl.ds(s * rps, rps), :],
                    agg.at[pl.ds(s * rps, rps), :])
    def addoff(g, carry):
      src_v[pl.ds(g * _L, _L)] = src_v[pl.ds(g * _L, _L)] + N
      return carry
    lax.fori_loop(0, cap // _L, addoff, 0)
    plsc.subcore_barrier()

    _stream_loop(table_hbm, agg, src_v, dst_v, buf0, buf1, sem0, sem1,
                 ssem0, ssem1, ncc, epw // _K)

    plsc.subcore_barrier()
    pltpu.sync_copy(agg.at[pl.ds(s * rps, rps), :],
                    out_hbm.at[1, c, pl.ds(s * rps, rps), :])

  return pl.kernel(
      body,
      out_type=jax.ShapeDtypeStruct((2, _NC, arows, D), jnp.float32),
      mesh=mesh,
      compiler_params=pltpu.CompilerParams(needs_layout_passes=False),
      scratch_types=[
          pltpu.VMEM((cap,), jnp.int32),
          pltpu.VMEM((cap,), jnp.int32),
          pltpu.VMEM((_L,), jnp.int32),
          pltpu.VMEM((_K, D), jnp.float32),
          pltpu.VMEM((_K, D), jnp.float32),
          pltpu.VMEM_SHARED((arows, D), jnp.float32),
          pltpu.SemaphoreType.DMA,
          pltpu.SemaphoreType.DMA,
          pltpu.SemaphoreType.DMA,
          pltpu.SemaphoreType.DMA,
      ],
  )


def _mlp1(x, agg, W1, b1, W2, b2, eps):
  N, d_in = x.shape
  d_h = W1.shape[1]
  nb = N // _BN

  def body(eps_ref, x_ref, a_ref, w1_ref, b1_ref, w2_ref, b2_ref, out_ref):
    t = (1.0 + eps_ref[0, 0]) * x_ref[...] + a_ref[...]
    u = jnp.dot(t, w1_ref[...], preferred_element_type=jnp.float32)
    u = jnp.maximum(u + b1_ref[...], 0.0)
    h = jnp.dot(u, w2_ref[...], preferred_element_type=jnp.float32)
    h = jnp.maximum(h + b2_ref[...], 0.0)
    out_ref[0] = h[:, :d_in]
    out_ref[1] = h[:, d_in:]

  return pl.pallas_call(
      body,
      grid=(nb,),
      in_specs=[
          pl.BlockSpec(memory_space=pltpu.SMEM),
          pl.BlockSpec((_BN, d_in), lambda i: (i, 0)),
          pl.BlockSpec((_BN, d_in), lambda i: (i, 0)),
          pl.BlockSpec((d_in, d_h), lambda i: (0, 0)),
          pl.BlockSpec((1, d_h), lambda i: (0, 0)),
          pl.BlockSpec((d_h, d_h), lambda i: (0, 0)),
          pl.BlockSpec((1, d_h), lambda i: (0, 0)),
      ],
      out_specs=pl.BlockSpec((2, _BN, d_in), lambda i: (0, i, 0)),
      out_shape=jax.ShapeDtypeStruct((2, N, d_in), jnp.float32),
      compiler_params=pltpu.CompilerParams(
          dimension_semantics=("arbitrary",)),
  )(eps, x, agg, W1, b1, W2, b2)


def _mlp2_pool(h_a, h_b, agg_a, agg_b, batch3d, W3, b3, W4, b4, eps,
               W5, b5):
  N, d_in = h_a.shape
  d_h = W3.shape[1]
  d_out = W5.shape[1]
  nb = N // _BN

  def body(eps_ref, ha_ref, hb_ref, aa_ref, ab_ref, batch_ref, w3_ref,
           b3_ref, w4_ref, b4_ref, w5_ref, b5_ref, out_ref, acc, cnt):
    i = pl.program_id(0)

    @pl.when(i == 0)
    def _():
      acc[...] = jnp.zeros_like(acc)
      cnt[...] = jnp.zeros_like(cnt)

    e = 1.0 + eps_ref[0, 0]
    ta = e * ha_ref[...] + aa_ref[...]
    tb = e * hb_ref[...] + ab_ref[...]
    t = jnp.concatenate([ta, tb], axis=1)
    u = jnp.dot(t, w3_ref[...], preferred_element_type=jnp.float32)
    u = jnp.maximum(u + b3_ref[...], 0.0)
    v = jnp.dot(u, w4_ref[...], preferred_element_type=jnp.float32)
    v = jnp.maximum(v + b4_ref[...], 0.0)
    bb = batch_ref[0, 0, :].reshape(_BN, 1)
    oh = (bb == lax.broadcasted_iota(jnp.int32, (_BN, _G), 1)
          ).astype(jnp.float32)
    acc[...] += lax.dot_general(oh, v, (((0,), (0,)), ((), ())),
                                preferred_element_type=jnp.float32)
    cnt[...] += jnp.sum(oh, axis=0, keepdims=True)

    @pl.when(i == nb - 1)
    def _():
      denom = jnp.maximum(cnt[...], 1.0).reshape(_G, 1)
      pooled = acc[...] / denom
      out_ref[...] = jnp.dot(pooled, w5_ref[...],
                             preferred_element_type=jnp.float32) + b5_ref[...]

  return pl.pallas_call(
      body,
      grid=(nb,),
      in_specs=[
          pl.BlockSpec(memory_space=pltpu.SMEM),
          pl.BlockSpec((_BN, d_in), lambda i: (i, 0)),
          pl.BlockSpec((_BN, d_in), lambda i: (i, 0)),
          pl.BlockSpec((_BN, d_in), lambda i: (i, 0)),
          pl.BlockSpec((_BN, d_in), lambda i: (i, 0)),
          pl.BlockSpec((1, 1, _BN), lambda i: (i, 0, 0)),
          pl.BlockSpec((2 * d_in, d_h), lambda i: (0, 0)),
          pl.BlockSpec((1, d_h), lambda i: (0, 0)),
          pl.BlockSpec((d_h, d_h), lambda i: (0, 0)),
          pl.BlockSpec((1, d_h), lambda i: (0, 0)),
          pl.BlockSpec((d_h, d_out), lambda i: (0, 0)),
          pl.BlockSpec((1, d_out), lambda i: (0, 0)),
      ],
      out_specs=pl.BlockSpec((_G, d_out), lambda i: (0, 0)),
      out_shape=jax.ShapeDtypeStruct((_G, d_out), jnp.float32),
      scratch_shapes=[
          pltpu.VMEM((_G, d_h), jnp.float32),
          pltpu.VMEM((1, _G), jnp.float32),
      ],
      compiler_params=pltpu.CompilerParams(
          dimension_semantics=("arbitrary",)),
  )(eps, h_a, h_b, agg_a, agg_b, batch3d, W3, b3, W4, b4, W5, b5)


def kernel(x, edge_index, batch, W1, b1, W2, b2, eps1, W3, b3, W4, b4,
           eps2, W5, b5):
  N, d_in = x.shape
  E = edge_index.shape[1]
  half = _half(N)

  src = edge_index[0]
  dst = edge_index[1]
  zeros = jnp.zeros((_arows(N), d_in), jnp.float32)
  src_dup = jnp.concatenate([src, src])

  def glue(o):
    return jnp.concatenate([o[0, :half], o[1, :N - half]], axis=0)

  o1, srcc, dstc, cntc = _make_seg_compact(N, d_in, E)(
      src_dup, dst, x, zeros)
  h1 = _mlp1(x, glue(o1), W1, b1.reshape(1, -1), W2, b2.reshape(1, -1),
             eps1.reshape(1, 1))

  table2 = h1.reshape(2 * N, d_in)
  o2 = _make_seg_reuse2(N, d_in, E)(srcc, dstc, cntc, table2, zeros)
  o2a, o2b = o2[0], o2[1]

  batch3d = batch.reshape(N // _BN, 1, _BN)
  out = _mlp2_pool(h1[0], h1[1], glue(o2a), glue(o2b), batch3d, W3,
                   b3.reshape(1, -1), W4, b4.reshape(1, -1),
                   eps2.reshape(1, 1), W5, b5.reshape(1, -1))
  return out


# speedup vs baseline: 1.0056x; 1.0056x over previous
import jax
import jax.numpy as jnp
from jax import lax
from jax.experimental import pallas as pl
from jax.experimental.pallas import tpu as pltpu
from jax.experimental.pallas import tpu_sc as plsc

_NC = 2
_NS = 16
_L = 16
_K = 80
_TR = 512
_G = 64
_BN = 1000


def _half(N):
  return -(-N // 16) * 8


def _arows(N):
  return -(-(_half(N) + _TR) // (_NS * 8)) * (_NS * 8)


def _seg_common(N, E):
  epw = E // _NS
  cap = epw + _K
  half = _half(N)
  arows = _arows(N)
  rps = arows // _NS
  return epw, cap, half, arows, rps


def _stream_loop(table_hbm, agg, src_v, dst_v, buf0, buf1, sem0, sem1,
                 ssem0, ssem1, ncc, nloop):
  def gstart(i, buf, sem):
    pltpu.async_copy(table_hbm.at[src_v.at[pl.ds(i * _K, _K)]], buf, sem)

  def gwait(i, buf, sem):
    pltpu.make_async_copy(table_hbm.at[src_v.at[pl.ds(i * _K, _K)]],
                          buf, sem).wait()

  def sstart(i, buf, sem):
    pltpu.async_copy(buf, agg.at[dst_v.at[pl.ds(i * _K, _K)]], sem,
                     add=True)

  def swait(i, buf, sem):
    pltpu.make_async_copy(buf, agg.at[dst_v.at[pl.ds(i * _K, _K)]],
                          sem).wait()

  def step(i, buf, sem, ssem, nbuf, nsem, nssem):
    @pl.when(i >= 1)
    def _():
      swait(i - 1, nbuf, nssem)

    @pl.when(i + 1 < ncc)
    def _():
      gstart(i + 1, nbuf, nsem)
    gwait(i, buf, sem)
    sstart(i, buf, ssem)

  @pl.when(ncc > 0)
  def _():
    gstart(0, buf0, sem0)

  def loop_body(i, carry):
    @pl.when(jnp.logical_and(i < ncc, i % 2 == 0))
    def _():
      step(i, buf0, sem0, ssem0, buf1, sem1, ssem1)

    @pl.when(jnp.logical_and(i < ncc, i % 2 == 1))
    def _():
      step(i, buf1, sem1, ssem1, buf0, sem0, ssem0)
    return carry

  lax.fori_loop(0, nloop, loop_body, 0)

  @pl.when(jnp.logical_and(ncc > 0, (ncc - 1) % 2 == 0))
  def _():
    swait(ncc - 1, buf0, ssem0)

  @pl.when(jnp.logical_and(ncc > 0, (ncc - 1) % 2 == 1))
  def _():
    swait(ncc - 1, buf1, ssem1)


def _make_seg_compact(N, D, E):
  epw, cap, half, arows, rps = _seg_common(N, E)
  assert epw % _K == 0 and _K % _L == 0

  mesh = plsc.VectorSubcoreMesh(core_axis_name="c", subcore_axis_name="s",
                                num_cores=_NC, num_subcores=_NS)

  def body(src_hbm, dst_hbm, table_hbm, zeros_hbm,
           out_hbm, srcc_hbm, dstc_hbm, cntc_hbm,
           src_v, dst_v, cnt_v, buf0, buf1, agg, sem0, sem1, ssem0,
           ssem1):
    c = lax.axis_index("c")
    s = lax.axis_index("s")
    lane = lax.iota(jnp.int32, _L)
    pltpu.sync_copy(zeros_hbm.at[pl.ds(s * rps, rps), :],
                    agg.at[pl.ds(s * rps, rps), :])
    pltpu.sync_copy(src_hbm.at[pl.ds(c * E + s * epw, epw)],
                    src_v.at[pl.ds(0, epw)])
    pltpu.sync_copy(dst_hbm.at[pl.ds(s * epw, epw)],
                    dst_v.at[pl.ds(0, epw)])

    lo = c * half
    def compact(g, cnt):
      d = dst_v[pl.ds(g * _L, _L)] - lo
      ok = jnp.logical_and(d >= 0, d < half)
      oki = jnp.where(ok, 1, 0)
      srcs = src_v[pl.ds(g * _L, _L)]
      packed = (srcs << 13) | jnp.where(ok, d, 0)
      sv = plsc.sort_key_val(oki, packed, descending=True)[1]
      src_v[pl.ds(cnt, _L)] = sv >> 13
      dst_v[pl.ds(cnt, _L)] = sv & ((1 << 13) - 1)
      return cnt + jnp.sum(oki)
    cnt = lax.fori_loop(0, epw // _L, compact, 0)
    ncc = (cnt + _K - 1) // _K

    def tailfix(t, carry):
      src_v[pl.ds(cnt + t * _L, _L)] = jnp.zeros((_L,), jnp.int32)
      dst_v[pl.ds(cnt + t * _L, _L)] = half + t * _L + lane
      return carry
    lax.fori_loop(0, _K // _L, tailfix, 0)

    pltpu.sync_copy(src_v, srcc_hbm.at[c, s])
    pltpu.sync_copy(dst_v, dstc_hbm.at[c, s])
    cnt_v[...] = jnp.full((_L,), cnt, jnp.int32)
    pltpu.sync_copy(cnt_v, cntc_hbm.at[c, s])
    plsc.subcore_barrier()

    _stream_loop(table_hbm, agg, src_v, dst_v, buf0, buf1, sem0, sem1,
                 ssem0, ssem1, ncc, epw // _K)

    plsc.subcore_barrier()
    pltpu.sync_copy(agg.at[pl.ds(s * rps, rps), :],
                    out_hbm.at[c, pl.ds(s * rps, rps), :])

  return pl.kernel(
      body,
      out_type=(
          jax.ShapeDtypeStruct((_NC, arows, D), jnp.float32),
          jax.ShapeDtypeStruct((_NC, _NS, cap), jnp.int32),
          jax.ShapeDtypeStruct((_NC, _NS, cap), jnp.int32),
          jax.ShapeDtypeStruct((_NC, _NS, _L), jnp.int32),
      ),
      mesh=mesh,
      compiler_params=pltpu.CompilerParams(needs_layout_passes=False),
      scratch_types=[
          pltpu.VMEM((cap,), jnp.int32),
          pltpu.VMEM((cap,), jnp.int32),
          pltpu.VMEM((_L,), jnp.int32),
          pltpu.VMEM((_K, D), jnp.float32),
          pltpu.VMEM((_K, D), jnp.float32),
          pltpu.VMEM_SHARED((arows, D), jnp.float32),
          pltpu.SemaphoreType.DMA,
          pltpu.SemaphoreType.DMA,
          pltpu.SemaphoreType.DMA,
          pltpu.SemaphoreType.DMA,
      ],
  )


def _make_seg_reuse(N, D, E, row_offset):
  epw, cap, half, arows, rps = _seg_common(N, E)

  mesh = plsc.VectorSubcoreMesh(core_axis_name="c", subcore_axis_name="s",
                                num_cores=_NC, num_subcores=_NS)

  def body(srcc_hbm, dstc_hbm, cntc_hbm, table_hbm, zeros_hbm, out_hbm,
           src_v, dst_v, cnt_v, buf0, buf1, agg, sem0, sem1, ssem0,
           ssem1):
    c = lax.axis_index("c")
    s = lax.axis_index("s")
    pltpu.sync_copy(zeros_hbm.at[pl.ds(s * rps, rps), :],
                    agg.at[pl.ds(s * rps, rps), :])
    pltpu.sync_copy(srcc_hbm.at[c, s], src_v)
    pltpu.sync_copy(dstc_hbm.at[c, s], dst_v)
    pltpu.sync_copy(cntc_hbm.at[c, s], cnt_v)
    cnt = lax.reduce_max(cnt_v[...], (0,))
    ncc = (cnt + _K - 1) // _K
    if row_offset:
      def addoff(g, carry):
        src_v[pl.ds(g * _L, _L)] = src_v[pl.ds(g * _L, _L)] + row_offset
        return carry
      lax.fori_loop(0, cap // _L, addoff, 0)
    plsc.subcore_barrier()

    _stream_loop(table_hbm, agg, src_v, dst_v, buf0, buf1, sem0, sem1,
                 ssem0, ssem1, ncc, epw // _K)

    plsc.subcore_barrier()
    pltpu.sync_copy(agg.at[pl.ds(s * rps, rps), :],
                    out_hbm.at[c, pl.ds(s * rps, rps), :])

  return pl.kernel(
      body,
      out_type=jax.ShapeDtypeStruct((_NC, arows, D), jnp.float32),
      mesh=mesh,
      compiler_params=pltpu.CompilerParams(needs_layout_passes=False),
      scratch_types=[
          pltpu.VMEM((cap,), jnp.int32),
          pltpu.VMEM((cap,), jnp.int32),
          pltpu.VMEM((_L,), jnp.int32),
          pltpu.VMEM((_K, D), jnp.float32),
          pltpu.VMEM((_K, D), jnp.float32),
          pltpu.VMEM_SHARED((arows, D), jnp.float32),
          pltpu.SemaphoreType.DMA,
          pltpu.SemaphoreType.DMA,
          pltpu.SemaphoreType.DMA,
          pltpu.SemaphoreType.DMA,
      ],
  )


def _mlp1(x, agg, W1, b1, W2, b2, eps):
  N, d_in = x.shape
  d_h = W1.shape[1]
  nb = N // _BN

  def body(eps_ref, x_ref, a_ref, w1_ref, b1_ref, w2_ref, b2_ref, out_ref):
    t = (1.0 + eps_ref[0, 0]) * x_ref[...] + a_ref[...]
    u = jnp.dot(t, w1_ref[...], preferred_element_type=jnp.float32)
    u = jnp.maximum(u + b1_ref[...], 0.0)
    h = jnp.dot(u, w2_ref[...], preferred_element_type=jnp.float32)
    h = jnp.maximum(h + b2_ref[...], 0.0)
    out_ref[0] = h[:, :d_in]
    out_ref[1] = h[:, d_in:]

  return pl.pallas_call(
      body,
      grid=(nb,),
      in_specs=[
          pl.BlockSpec(memory_space=pltpu.SMEM),
          pl.BlockSpec((_BN, d_in), lambda i: (i, 0)),
          pl.BlockSpec((_BN, d_in), lambda i: (i, 0)),
          pl.BlockSpec((d_in, d_h), lambda i: (0, 0)),
          pl.BlockSpec((1, d_h), lambda i: (0, 0)),
          pl.BlockSpec((d_h, d_h), lambda i: (0, 0)),
          pl.BlockSpec((1, d_h), lambda i: (0, 0)),
      ],
      out_specs=pl.BlockSpec((2, _BN, d_in), lambda i: (0, i, 0)),
      out_shape=jax.ShapeDtypeStruct((2, N, d_in), jnp.float32),
      compiler_params=pltpu.CompilerParams(
          dimension_semantics=("arbitrary",)),
  )(eps, x, agg, W1, b1, W2, b2)


def _mlp2_pool(h_a, h_b, agg_a, agg_b, batch3d, W3, b3, W4, b4, eps,
               W5, b5):
  N, d_in = h_a.shape
  d_h = W3.shape[1]
  d_out = W5.shape[1]
  nb = N // _BN

  def body(eps_ref, ha_ref, hb_ref, aa_ref, ab_ref, batch_ref, w3_ref,
           b3_ref, w4_ref, b4_ref, w5_ref, b5_ref, out_ref, acc, cnt):
    i = pl.program_id(0)

    @pl.when(i == 0)
    def _():
      acc[...] = jnp.zeros_like(acc)
      cnt[...] = jnp.zeros_like(cnt)

    e = 1.0 + eps_ref[0, 0]
    ta = e * ha_ref[...] + aa_ref[...]
    tb = e * hb_ref[...] + ab_ref[...]
    t = jnp.concatenate([ta, tb], axis=1)
    u = jnp.dot(t, w3_ref[...], preferred_element_type=jnp.float32)
    u = jnp.maximum(u + b3_ref[...], 0.0)
    v = jnp.dot(u, w4_ref[...], preferred_element_type=jnp.float32)
    v = jnp.maximum(v + b4_ref[...], 0.0)
    bb = batch_ref[0, 0, :].reshape(_BN, 1)
    oh = (bb == lax.broadcasted_iota(jnp.int32, (_BN, _G), 1)
          ).astype(jnp.float32)
    acc[...] += lax.dot_general(oh, v, (((0,), (0,)), ((), ())),
                                preferred_element_type=jnp.float32)
    cnt[...] += jnp.sum(oh, axis=0, keepdims=True)

    @pl.when(i == nb - 1)
    def _():
      denom = jnp.maximum(cnt[...], 1.0).reshape(_G, 1)
      pooled = acc[...] / denom
      out_ref[...] = jnp.dot(pooled, w5_ref[...],
                             preferred_element_type=jnp.float32) + b5_ref[...]

  return pl.pallas_call(
      body,
      grid=(nb,),
      in_specs=[
          pl.BlockSpec(memory_space=pltpu.SMEM),
          pl.BlockSpec((_BN, d_in), lambda i: (i, 0)),
          pl.BlockSpec((_BN, d_in), lambda i: (i, 0)),
          pl.BlockSpec((_BN, d_in), lambda i: (i, 0)),
          pl.BlockSpec((_BN, d_in), lambda i: (i, 0)),
          pl.BlockSpec((1, 1, _BN), lambda i: (i, 0, 0)),
          pl.BlockSpec((2 * d_in, d_h), lambda i: (0, 0)),
          pl.BlockSpec((1, d_h), lambda i: (0, 0)),
          pl.BlockSpec((d_h, d_h), lambda i: (0, 0)),
          pl.BlockSpec((1, d_h), lambda i: (0, 0)),
          pl.BlockSpec((d_h, d_out), lambda i: (0, 0)),
          pl.BlockSpec((1, d_out), lambda i: (0, 0)),
      ],
      out_specs=pl.BlockSpec((_G, d_out), lambda i: (0, 0)),
      out_shape=jax.ShapeDtypeStruct((_G, d_out), jnp.float32),
      scratch_shapes=[
          pltpu.VMEM((_G, d_h), jnp.float32),
          pltpu.VMEM((1, _G), jnp.float32),
      ],
      compiler_params=pltpu.CompilerParams(
          dimension_semantics=("arbitrary",)),
  )(eps, h_a, h_b, agg_a, agg_b, batch3d, W3, b3, W4, b4, W5, b5)


def kernel(x, edge_index, batch, W1, b1, W2, b2, eps1, W3, b3, W4, b4,
           eps2, W5, b5):
  N, d_in = x.shape
  E = edge_index.shape[1]
  half = _half(N)

  src = edge_index[0]
  dst = edge_index[1]
  zeros = jnp.zeros((_arows(N), d_in), jnp.float32)
  src_dup = jnp.concatenate([src, src])

  def glue(o):
    return jnp.concatenate([o[0, :half], o[1, :N - half]], axis=0)

  o1, srcc, dstc, cntc = _make_seg_compact(N, d_in, E)(
      src_dup, dst, x, zeros)
  h1 = _mlp1(x, glue(o1), W1, b1.reshape(1, -1), W2, b2.reshape(1, -1),
             eps1.reshape(1, 1))

  table2 = h1.reshape(2 * N, d_in)
  o2a = _make_seg_reuse(N, d_in, E, 0)(srcc, dstc, cntc, table2, zeros)
  o2b = _make_seg_reuse(N, d_in, E, N)(srcc, dstc, cntc, table2, zeros)

  batch3d = batch.reshape(N // _BN, 1, _BN)
  out = _mlp2_pool(h1[0], h1[1], glue(o2a), glue(o2b), batch3d, W3,
                   b3.reshape(1, -1), W4, b4.reshape(1, -1),
                   eps2.reshape(1, 1), W5, b5.reshape(1, -1))
  return out
